# Initial kernel scaffold; baseline (speedup 1.0000x reference)
import jax
import jax.numpy as jnp
from jax.experimental import pallas as pl


def kernel(x, w1, w2):
    raise NotImplementedError("write your pallas kernel here")


# speedup vs baseline: 1.2654x; 1.2654x over previous
import functools

import jax
import jax.numpy as jnp
from jax.experimental import pallas as pl
from jax.experimental.pallas import tpu as pltpu


def _round_up(x, m):
    return (x + m - 1) // m * m


def _se_fused_kernel(x_ref, w1t_ref, w2t_ref, o_ref, *, inv_hw):
    x = x_ref[...].astype(jnp.float32)
    pooled = jnp.sum(x, axis=-1) * inv_hw
    y1 = jnp.dot(pooled, w1t_ref[...], preferred_element_type=jnp.float32)
    y1 = jnp.maximum(y1, 0.0)
    y2 = jnp.dot(y1, w2t_ref[...], preferred_element_type=jnp.float32)
    gate = jax.nn.sigmoid(y2)
    o_ref[...] = (x * gate[:, :, None]).astype(o_ref.dtype)


def kernel(x, w1, w2):
    B, C, H, W = x.shape
    HW = H * W
    hid = w1.shape[0]

    hw_pad = _round_up(HW, 128)
    x_flat = x.reshape(B, C, HW)
    if hw_pad != HW:
        x_flat = jnp.pad(x_flat, ((0, 0), (0, 0), (0, hw_pad - HW)))

    w1t = w1.T.astype(jnp.float32)
    w2t = w2.T.astype(jnp.float32)

    bb = 1
    out_flat = pl.pallas_call(
        functools.partial(_se_fused_kernel, inv_hw=1.0 / float(HW)),
        out_shape=jax.ShapeDtypeStruct((B, C, hw_pad), x.dtype),
        grid=(B // bb,),
        in_specs=[
            pl.BlockSpec((bb, C, hw_pad), lambda b: (b, 0, 0)),
            pl.BlockSpec((C, hid), lambda b: (0, 0)),
            pl.BlockSpec((hid, C), lambda b: (0, 0)),
        ],
        out_specs=pl.BlockSpec((bb, C, hw_pad), lambda b: (b, 0, 0)),
        compiler_params=pltpu.CompilerParams(
            dimension_semantics=("parallel",)),
    )(x_flat, w1t, w2t)

    if hw_pad != HW:
        out_flat = out_flat[:, :, :HW]
    return out_flat.reshape(B, C, H, W)


# speedup vs baseline: 1.4943x; 1.1809x over previous
import functools

import jax
import jax.numpy as jnp
from jax.experimental import pallas as pl
from jax.experimental.pallas import tpu as pltpu


def _round_up(x, m):
    return (x + m - 1) // m * m


def _se_fused_kernel(x_ref, w1t_ref, w2t_ref, o_ref, *, inv_hw):
    x = x_ref[...].astype(jnp.float32)
    pooled = jnp.sum(x, axis=-1) * inv_hw
    y1 = jnp.dot(pooled, w1t_ref[...], preferred_element_type=jnp.float32)
    y1 = jnp.maximum(y1, 0.0)
    y2 = jnp.dot(y1, w2t_ref[...], preferred_element_type=jnp.float32)
    gate = jax.nn.sigmoid(y2)
    o_ref[...] = (x * gate[:, :, None]).astype(o_ref.dtype)


def kernel(x, w1, w2):
    B, C, H, W = x.shape
    HW = H * W
    hid = w1.shape[0]

    hw_pad = _round_up(HW, 128)
    x_flat = x.reshape(B, C, HW)
    if hw_pad != HW:
        x_flat = jnp.pad(x_flat, ((0, 0), (0, 0), (0, hw_pad - HW)))

    w1t = w1.T.astype(jnp.float32)
    w2t = w2.T.astype(jnp.float32)

    bb = 4
    out_flat = pl.pallas_call(
        functools.partial(_se_fused_kernel, inv_hw=1.0 / float(HW)),
        out_shape=jax.ShapeDtypeStruct((B, C, hw_pad), x.dtype),
        grid=(B // bb,),
        in_specs=[
            pl.BlockSpec((bb, C, hw_pad), lambda b: (b, 0, 0)),
            pl.BlockSpec((C, hid), lambda b: (0, 0)),
            pl.BlockSpec((hid, C), lambda b: (0, 0)),
        ],
        out_specs=pl.BlockSpec((bb, C, hw_pad), lambda b: (b, 0, 0)),
        compiler_params=pltpu.CompilerParams(
            dimension_semantics=("parallel",)),
    )(x_flat, w1t, w2t)

    if hw_pad != HW:
        out_flat = out_flat[:, :, :HW]
    return out_flat.reshape(B, C, H, W)


# speedup vs baseline: 1.5190x; 1.0166x over previous
import functools

import jax
import jax.numpy as jnp
from jax.experimental import pallas as pl
from jax.experimental.pallas import tpu as pltpu


def _round_up(x, m):
    return (x + m - 1) // m * m


def _se_fused_kernel(x_ref, w1t_ref, w2t_ref, o_ref, *, inv_hw):
    x = x_ref[...].astype(jnp.float32)
    pooled = jnp.sum(x, axis=-1) * inv_hw
    y1 = jnp.dot(pooled, w1t_ref[...], preferred_element_type=jnp.float32)
    y1 = jnp.maximum(y1, 0.0)
    y2 = jnp.dot(y1, w2t_ref[...], preferred_element_type=jnp.float32)
    gate = jax.nn.sigmoid(y2)
    o_ref[...] = (x * gate[:, :, None]).astype(o_ref.dtype)


def kernel(x, w1, w2):
    B, C, H, W = x.shape
    HW = H * W
    hid = w1.shape[0]

    hw_pad = _round_up(HW, 128)
    x_flat = x.reshape(B, C, HW)
    if hw_pad != HW:
        x_flat = jnp.pad(x_flat, ((0, 0), (0, 0), (0, hw_pad - HW)))

    w1t = w1.T.astype(jnp.float32)
    w2t = w2.T.astype(jnp.float32)

    bb = 8
    out_flat = pl.pallas_call(
        functools.partial(_se_fused_kernel, inv_hw=1.0 / float(HW)),
        out_shape=jax.ShapeDtypeStruct((B, C, hw_pad), x.dtype),
        grid=(B // bb,),
        in_specs=[
            pl.BlockSpec((bb, C, hw_pad), lambda b: (b, 0, 0)),
            pl.BlockSpec((C, hid), lambda b: (0, 0)),
            pl.BlockSpec((hid, C), lambda b: (0, 0)),
        ],
        out_specs=pl.BlockSpec((bb, C, hw_pad), lambda b: (b, 0, 0)),
        compiler_params=pltpu.CompilerParams(
            dimension_semantics=("parallel",)),
    )(x_flat, w1t, w2t)

    if hw_pad != HW:
        out_flat = out_flat[:, :, :HW]
    return out_flat.reshape(B, C, H, W)


# speedup vs baseline: 1.5217x; 1.0017x over previous
import functools

import jax
import jax.numpy as jnp
from jax.experimental import pallas as pl
from jax.experimental.pallas import tpu as pltpu


def _round_up(x, m):
    return (x + m - 1) // m * m


def _se_fused_kernel(x_ref, w1_ref, w2_ref, o_ref, *, inv_hw):
    x = x_ref[...].astype(jnp.float32)
    pooled = jnp.sum(x, axis=-1) * inv_hw
    y1 = jax.lax.dot_general(
        pooled, w1_ref[...], (((1,), (1,)), ((), ())),
        preferred_element_type=jnp.float32)
    y1 = jnp.maximum(y1, 0.0)
    y2 = jax.lax.dot_general(
        y1, w2_ref[...], (((1,), (1,)), ((), ())),
        preferred_element_type=jnp.float32)
    gate = jax.nn.sigmoid(y2)
    o_ref[...] = (x * gate[:, :, None]).astype(o_ref.dtype)


def kernel(x, w1, w2):
    B, C, H, W = x.shape
    HW = H * W
    hid = w1.shape[0]

    hw_pad = _round_up(HW, 128)
    x_flat = x.reshape(B, C, HW)
    if hw_pad != HW:
        x_flat = jnp.pad(x_flat, ((0, 0), (0, 0), (0, hw_pad - HW)))

    slab_bytes = C * hw_pad * jnp.dtype(x.dtype).itemsize
    bb = B
    while bb > 1 and (bb * slab_bytes * 4 > 48 * 1024 * 1024 or B % bb):
        bb -= 1

    out_flat = pl.pallas_call(
        functools.partial(_se_fused_kernel, inv_hw=1.0 / float(HW)),
        out_shape=jax.ShapeDtypeStruct((B, C, hw_pad), x.dtype),
        grid=(B // bb,),
        in_specs=[
            pl.BlockSpec((bb, C, hw_pad), lambda b: (b, 0, 0)),
            pl.BlockSpec((hid, C), lambda b: (0, 0)),
            pl.BlockSpec((C, hid), lambda b: (0, 0)),
        ],
        out_specs=pl.BlockSpec((bb, C, hw_pad), lambda b: (b, 0, 0)),
        compiler_params=pltpu.CompilerParams(
            dimension_semantics=("parallel",)),
    )(x_flat, w1, w2)

    if hw_pad != HW:
        out_flat = out_flat[:, :, :HW]
    return out_flat.reshape(B, C, H, W)
